# baseline (speedup 1.0000x reference)
import functools

import jax
import jax.numpy as jnp
from jax import lax
from jax.experimental import pallas as pl
from jax.experimental.pallas import tpu as pltpu
from jax.experimental.pallas import tpu_sc as plsc

N = 10000
NP = 10008
DX = 128
DH = 64
DRL = 16
DG = 256
NREL = 237
NRELP = 240
NE = 160000
ALPHA = 0.2
NC = 2
NS = 16
NW = NC * NS
EPW = 5120
EPAD = EPW * NW
CH = 64
NCHUNK = EPW // CH
ZR = 24
NZC = 26
NR = NZC * ZR
NTAIL = NP - NR * NS


def _elu(v):
    return jnp.where(v > 0, v, jnp.exp(v) - 1.0)


def _splat(vec16, lidx):
    return lax.gather(
        vec16, lidx[:, None],
        lax.GatherDimensionNumbers(offset_dims=(), collapsed_slice_dims=(0,),
                                   start_index_map=(0,)),
        (1,), mode=lax.GatherScatterMode.PROMISE_IN_BOUNDS)


def _colvec(m, v2):
    return lax.dot_general(m, v2, (((1,), (1,)), ((), ())),
                           preferred_element_type=jnp.float32)


def _tc_pre(x_ref, relp_ref, wt_ref, ws_ref,
            a0ta_ref, a1ta_ref, a0tb_ref, a1tb_ref, a2ta_ref, a2tb_ref,
            v2a_ref, v2b_ref, w1_ref, a2lt_ref, a2l_ref, w3_ref,
            ete_ref, ese_ref, u0cat_ref, u1s_ref, ptg_ref,
            rext_ref, rlext_ref, orf_ref):
    x = x_ref[...]
    f32 = jnp.float32
    ete_ref[...] = jnp.dot(x, wt_ref[...], preferred_element_type=f32)
    ese_ref[...] = jnp.dot(x, ws_ref[...], preferred_element_type=f32)
    u0a = jnp.dot(x, a0ta_ref[...], preferred_element_type=f32)
    u1a = jnp.dot(x, a1ta_ref[...], preferred_element_type=f32)
    u0b = jnp.dot(x, a0tb_ref[...], preferred_element_type=f32)
    u1b = jnp.dot(x, a1tb_ref[...], preferred_element_type=f32)
    v2a = v2a_ref[...]
    v2b = v2b_ref[...]
    u0cat_ref[...] = jnp.concatenate([u0a, u0b], axis=1)
    u1s_ref[0:N, :] = jnp.concatenate(
        [u1a, u1b, _colvec(u1a, v2a), _colvec(u1b, v2b),
         jnp.zeros((N, DG - DX - 2), f32)], axis=1)
    u1s_ref[N:NP, :] = jnp.zeros((NP - N, DG), f32)
    ptg_ref[0:N, :] = jnp.concatenate(
        [_colvec(u0a, v2a), _colvec(u0b, v2b), jnp.zeros((N, DX - 2), f32)],
        axis=1)
    ptg_ref[N:NP, :] = jnp.zeros((NP - N, DX), f32)
    relp = relp_ref[...]
    ra = jnp.dot(relp, a2ta_ref[...], preferred_element_type=f32)
    rb = jnp.dot(relp, a2tb_ref[...], preferred_element_type=f32)
    rext_ref[...] = jnp.concatenate(
        [ra, rb, _colvec(ra, v2a), _colvec(rb, v2b),
         jnp.zeros((NRELP, DG - DX - 2), f32)], axis=1)
    o1 = jnp.dot(relp, w1_ref[...], preferred_element_type=f32)
    rl = jnp.dot(o1, a2lt_ref[...], preferred_element_type=f32)
    a2l = a2l_ref[...]
    rlext_ref[...] = jnp.concatenate(
        [rl, _colvec(rl, a2l), jnp.zeros((NRELP, DG - DX - 1), f32)], axis=1)
    orf_ref[...] = jnp.dot(o1, w3_ref[...], preferred_element_type=f32)


def _tc_mid(u0cat_ref, tcat_ref, sa_ref, sb_ref, ct_ref, cs_ref, ese_ref,
            a0lt_ref, a1lt_ref, a2l_ref,
            u0l_ref, u1sl_ref, ptgl_ref, tmask_ref):
    f32 = jnp.float32
    sa = sa_ref[0, :] + sa_ref[1, :]
    sb = sb_ref[0, :] + sb_ref[1, :]
    t = tcat_ref[0][0:N, :] + tcat_ref[1][0:N, :]
    scat = jnp.concatenate(
        [jnp.broadcast_to(sa[:, None], (N, DH)),
         jnp.broadcast_to(sb[:, None], (N, DH))], axis=1)
    h = (u0cat_ref[...] * scat + t) / (scat + 1e-12)
    x = _elu(h)
    ct = ct_ref[0, :] + ct_ref[1, :]
    cs = cs_ref[0, :] + cs_ref[1, :]
    tmask = ct > 0.5
    srcof = jnp.where(jnp.logical_and(cs > 0.5, jnp.logical_not(tmask)), 1.0, 0.0)
    x = jnp.where(srcof[:, None] > 0.5, ese_ref[...], x)
    u0l = jnp.dot(x, a0lt_ref[...], preferred_element_type=f32)
    u1l = jnp.dot(x, a1lt_ref[...], preferred_element_type=f32)
    u0l_ref[...] = u0l
    a2l = a2l_ref[...]
    u1sl_ref[0:N, :] = jnp.concatenate(
        [u1l, _colvec(u1l, a2l), jnp.zeros((N, DG - DX - 1), f32)], axis=1)
    u1sl_ref[N:NP, :] = jnp.zeros((NP - N, DG), f32)
    ptgl_ref[0:N, :] = jnp.concatenate(
        [_colvec(u0l, a2l), jnp.zeros((N, DX - 1), f32)], axis=1)
    ptgl_ref[N:NP, :] = jnp.zeros((NP - N, DX), f32)
    tmask_ref[...] = jnp.where(tmask, 1.0, 0.0)[None, :]


def _tc_fin(x_ref, u0l_ref, tl_ref, sl_ref, tmask_ref, ete_ref, out_ref):
    sl = sl_ref[0, :] + sl_ref[1, :]
    t = tl_ref[0][0:N, :] + tl_ref[1][0:N, :]
    h = (u0l_ref[...] * sl[:, None] + t) / (sl + 1e-12)[:, None]
    xf = _elu(h)
    tm = tmask_ref[0, :]
    out_ref[...] = jnp.where(tm[:, None] > 0.5, xf + ete_ref[...], x_ref[...])


_MESH = plsc.VectorSubcoreMesh(core_axis_name="c", subcore_axis_name="s",
                               num_cores=NC)


def _edge_pass(nh, tgt_hbm, src_hbm, et_hbm, u1s_hbm, rext_hbm, ptg_hbm,
               t_out, s_outs, tacc, saccs, scr):
    c = lax.axis_index("c")
    s = lax.axis_index("s")
    wid = c * NS + s
    njh = DX // (16 * nh)
    (tgt_v, src_v, et_v, u1r, rr, scb, wa, wb, vf, zb, zb1,
     isem, gsem, ssem) = scr

    iota16 = lax.iota(jnp.int32, 16)
    l0 = iota16 * 0
    l1 = l0 + 1
    ones16 = l0.astype(jnp.float32) + 1.0

    def zrow(i, carry):
        for j in range(DX // 16):
            zb[i, pl.ds(16 * j, 16)] = jnp.zeros((16,), jnp.float32)
        return carry
    lax.fori_loop(0, ZR, zrow, 0)

    def zrow1(i, carry):
        zb1[pl.ds(16 * i, 16)] = jnp.zeros((16,), jnp.float32)
        return carry
    lax.fori_loop(0, 63, zrow1, 0)

    for k in range(NZC):
        pltpu.sync_copy(zb, tacc.at[pl.ds(NR * s + ZR * k, ZR)])

    @pl.when(s == 0)
    def _():
        pltpu.sync_copy(zb, tacc.at[pl.ds(NR * NS, NTAIL)])

    @pl.when(s < 10)
    def _():
        for a in saccs:
            pltpu.sync_copy(zb1.at[pl.ds(0, 1000)], a.at[pl.ds(1000 * s, 1000)])
    plsc.subcore_barrier()

    for g in range(CH // 16):
        vf[pl.ds(16 * g, 16)] = ones16

    nsc = len(saccs)

    def fire_scatters():
        pltpu.async_copy(scb, tacc.at[tgt_v], ssem, add=True)
        pltpu.async_copy(wa.at[pl.ds(0, CH)], saccs[0].at[tgt_v], ssem, add=True)
        if nh == 2:
            pltpu.async_copy(wb.at[pl.ds(0, CH)], saccs[1].at[tgt_v], ssem,
                             add=True)
            pltpu.async_copy(vf, saccs[2].at[tgt_v], ssem, add=True)
            pltpu.async_copy(vf, saccs[3].at[src_v], ssem, add=True)

    def wait_scatters():
        pltpu.make_async_copy(scb, tacc.at[tgt_v], ssem).wait()
        pltpu.make_async_copy(wa.at[pl.ds(0, CH)], saccs[0].at[tgt_v], ssem).wait()
        if nh == 2:
            pltpu.make_async_copy(wb.at[pl.ds(0, CH)], saccs[1].at[tgt_v],
                                  ssem).wait()
            pltpu.make_async_copy(vf, saccs[2].at[tgt_v], ssem).wait()
            pltpu.make_async_copy(vf, saccs[3].at[src_v], ssem).wait()

    def chunk_body(ci, carry):
        @pl.when(ci > 0)
        def _():
            wait_scatters()
        base = wid * EPW + ci * CH
        pltpu.async_copy(tgt_hbm.at[pl.ds(base, CH)], tgt_v, isem)
        pltpu.async_copy(src_hbm.at[pl.ds(base, CH)], src_v, isem)
        pltpu.async_copy(et_hbm.at[pl.ds(base, CH)], et_v, isem)
        pltpu.make_async_copy(tgt_hbm.at[pl.ds(0, CH)], tgt_v, isem).wait()
        pltpu.make_async_copy(src_hbm.at[pl.ds(0, CH)], src_v, isem).wait()
        pltpu.make_async_copy(et_hbm.at[pl.ds(0, CH)], et_v, isem).wait()
        pltpu.async_copy(u1s_hbm.at[src_v], u1r, gsem)
        pltpu.async_copy(rext_hbm.at[et_v], rr, gsem)
        pltpu.async_copy(ptg_hbm.at[tgt_v], scb, gsem)
        pltpu.make_async_copy(u1s_hbm.at[src_v], u1r, gsem).wait()
        pltpu.make_async_copy(rext_hbm.at[et_v], rr, gsem).wait()
        pltpu.make_async_copy(ptg_hbm.at[tgt_v], scb, gsem).wait()

        def edge_body(i, carry2):
            sv = (scb[i, pl.ds(0, 16)] + u1r[i, pl.ds(DX, 16)]
                  + rr[i, pl.ds(DX, 16)])
            pw = jnp.where(sv > 0, -sv, (-ALPHA) * sv)
            wv = jnp.exp(pw)
            spl = [_splat(wv, l0)]
            wa[pl.ds(i, 16)] = spl[0]
            if nh == 2:
                spl.append(_splat(wv, l1))
                wb[pl.ds(i, 16)] = spl[1]
            for h in range(nh):
                for j in range(njh):
                    jsl = pl.ds(16 * (h * njh + j), 16)
                    scb[i, jsl] = spl[h] * (u1r[i, jsl] + rr[i, jsl])
            return carry2

        lax.fori_loop(0, CH, edge_body, 0)
        fire_scatters()
        return carry

    lax.fori_loop(0, NCHUNK, chunk_body, 0)
    wait_scatters()
    plsc.subcore_barrier()

    for k in range(NZC):
        off = NR * s + ZR * k
        pltpu.sync_copy(tacc.at[pl.ds(off, ZR)], zb)
        pltpu.sync_copy(zb, t_out.at[c, pl.ds(off, ZR)])

    @pl.when(s == 0)
    def _():
        pltpu.sync_copy(tacc.at[pl.ds(NR * NS, NTAIL)], zb)
        pltpu.sync_copy(zb, t_out.at[c, pl.ds(NR * NS, NTAIL)])

    @pl.when(s < 10)
    def _():
        for a, o in zip(saccs, s_outs):
            pltpu.sync_copy(a.at[pl.ds(1000 * s, 1000)], zb1.at[pl.ds(0, 1000)])
            pltpu.sync_copy(zb1.at[pl.ds(0, 1000)],
                            o.at[pl.ds(c * N + 1000 * s, 1000)])


def _scratch_types(nsc):
    return ([pltpu.VMEM((CH,), jnp.int32)] * 3 +
            [pltpu.VMEM((CH, DG), jnp.float32)] * 2 +
            [pltpu.VMEM((CH, DX), jnp.float32)] +
            [pltpu.VMEM((CH + 16,), jnp.float32)] * 2 +
            [pltpu.VMEM((CH,), jnp.float32)] +
            [pltpu.VMEM((ZR, DX), jnp.float32),
             pltpu.VMEM((1008,), jnp.float32)] +
            [pltpu.SemaphoreType.DMA] * 3)


@functools.partial(
    pl.kernel,
    out_type=[jax.ShapeDtypeStruct((NC, NP, DX), jnp.float32)] +
             [jax.ShapeDtypeStruct((NC * N,), jnp.float32)] * 4,
    mesh=_MESH,
    scratch_types=[pltpu.VMEM_SHARED((NP, DX), jnp.float32)] +
                  [pltpu.VMEM_SHARED((NP,), jnp.float32)] * 4 +
                  _scratch_types(4),
)
def _sc_layer0(tgt_hbm, src_hbm, et_hbm, u1s_hbm, rext_hbm, ptg_hbm,
               t_out, sa_out, sb_out, ct_out, cs_out,
               tacc, sacc_a, sacc_b, ctacc, csacc, *scr):
    _edge_pass(2, tgt_hbm, src_hbm, et_hbm, u1s_hbm, rext_hbm, ptg_hbm,
               t_out, [sa_out, sb_out, ct_out, cs_out],
               tacc, [sacc_a, sacc_b, ctacc, csacc], list(scr))


@functools.partial(
    pl.kernel,
    out_type=[jax.ShapeDtypeStruct((NC, NP, DX), jnp.float32),
              jax.ShapeDtypeStruct((NC * N,), jnp.float32)],
    mesh=_MESH,
    scratch_types=[pltpu.VMEM_SHARED((NP, DX), jnp.float32),
                   pltpu.VMEM_SHARED((NP,), jnp.float32)] +
                  _scratch_types(1),
)
def _sc_final(tgt_hbm, src_hbm, et_hbm, u1s_hbm, rext_hbm, ptg_hbm,
              tl_out, sl_out, tacc, sacc, *scr):
    _edge_pass(1, tgt_hbm, src_hbm, et_hbm, u1s_hbm, rext_hbm, ptg_hbm,
               tl_out, [sl_out], tacc, [sacc], list(scr))


def kernel(entity_embeddings, relation_embed, edge_list, edge_type,
           W1, W3, W_source, W_target, a0, a2_0, a_last, a2_last):
    f32 = jnp.float32
    x = entity_embeddings.astype(f32)
    relp = jnp.zeros((NRELP, DRL), f32).at[:NREL].set(relation_embed.astype(f32))

    el = jnp.asarray(edge_list, jnp.int32)
    et = jnp.asarray(edge_type, jnp.int32)
    tgt = jnp.pad(el[0], (0, EPAD - NE), constant_values=N)
    src = jnp.pad(el[1], (0, EPAD - NE), constant_values=N)
    etp = jnp.pad(et, (0, EPAD - NE), constant_values=NREL)

    a0 = a0.astype(f32)
    a0ta = a0[0, :, :DX].T
    a1ta = a0[0, :, DX:2 * DX].T
    a2ta = a0[0, :, 2 * DX:].T
    a0tb = a0[1, :, :DX].T
    a1tb = a0[1, :, DX:2 * DX].T
    a2tb = a0[1, :, 2 * DX:].T
    v2a = a2_0[0].astype(f32)
    v2b = a2_0[1].astype(f32)
    a_last = a_last.astype(f32)
    a0lt = a_last[:, :DX].T
    a1lt = a_last[:, DX:2 * DX].T
    a2lt = a_last[:, 2 * DX:].T
    a2l = a2_last.astype(f32)

    shp = [
        jax.ShapeDtypeStruct((N, DX), f32),
        jax.ShapeDtypeStruct((N, DX), f32),
        jax.ShapeDtypeStruct((N, DX), f32),
        jax.ShapeDtypeStruct((NP, DG), f32),
        jax.ShapeDtypeStruct((NP, DX), f32),
        jax.ShapeDtypeStruct((NRELP, DG), f32),
        jax.ShapeDtypeStruct((NRELP, DG), f32),
        jax.ShapeDtypeStruct((NRELP, DX), f32),
    ]
    (ete, ese, u0cat, u1s, ptg, rext, rlext, orf) = pl.pallas_call(
        _tc_pre, out_shape=shp)(
        x, relp, W_target.astype(f32), W_source.astype(f32),
        a0ta, a1ta, a0tb, a1tb, a2ta, a2tb, v2a, v2b,
        W1.astype(f32), a2lt, a2l, W3.astype(f32))

    tcat, sa, sb, ct, cs = _sc_layer0(tgt, src, etp, u1s, rext, ptg)

    u0l, u1sl, ptgl, tmask = pl.pallas_call(
        _tc_mid,
        out_shape=[
            jax.ShapeDtypeStruct((N, DX), f32),
            jax.ShapeDtypeStruct((NP, DG), f32),
            jax.ShapeDtypeStruct((NP, DX), f32),
            jax.ShapeDtypeStruct((1, N), f32),
        ])(u0cat, tcat, sa.reshape(NC, N), sb.reshape(NC, N),
           ct.reshape(NC, N), cs.reshape(NC, N), ese, a0lt, a1lt, a2l)

    tl, sl = _sc_final(tgt, src, etp, u1sl, rlext, ptgl)

    new_emb = pl.pallas_call(
        _tc_fin, out_shape=jax.ShapeDtypeStruct((N, DX), f32))(
        x, u0l, tl, sl.reshape(NC, N), tmask, ete)

    return new_emb, orf[:NREL]


# speedup vs baseline: 1.3250x; 1.3250x over previous
import functools

import jax
import jax.numpy as jnp
from jax import lax
from jax.experimental import pallas as pl
from jax.experimental.pallas import tpu as pltpu
from jax.experimental.pallas import tpu_sc as plsc

N = 10000
NP = 10008
DX = 128
DH = 64
DRL = 16
DG = 256
NREL = 237
NRELP = 240
NE = 160000
ALPHA = 0.2
NC = 2
NS = 16
NW = NC * NS
EPW = 5120
EPAD = EPW * NW
CH = 32
NCHUNK = EPW // CH
ZR = 24
NZC = 26
NR = NZC * ZR
NTAIL = NP - NR * NS


def _elu(v):
    return jnp.where(v > 0, v, jnp.exp(v) - 1.0)


def _splat(vec16, lidx):
    return lax.gather(
        vec16, lidx[:, None],
        lax.GatherDimensionNumbers(offset_dims=(), collapsed_slice_dims=(0,),
                                   start_index_map=(0,)),
        (1,), mode=lax.GatherScatterMode.PROMISE_IN_BOUNDS)


def _colvec(m, v2):
    return lax.dot_general(m, v2, (((1,), (1,)), ((), ())),
                           preferred_element_type=jnp.float32)


def _tc_pre(x_ref, relp_ref, wt_ref, ws_ref,
            a0ta_ref, a1ta_ref, a0tb_ref, a1tb_ref, a2ta_ref, a2tb_ref,
            v2a_ref, v2b_ref, w1_ref, a2lt_ref, a2l_ref, w3_ref,
            ete_ref, ese_ref, u0cat_ref, u1s_ref, ptg_ref,
            rext_ref, rlext_ref, orf_ref):
    x = x_ref[...]
    f32 = jnp.float32
    ete_ref[...] = jnp.dot(x, wt_ref[...], preferred_element_type=f32)
    ese_ref[...] = jnp.dot(x, ws_ref[...], preferred_element_type=f32)
    u0a = jnp.dot(x, a0ta_ref[...], preferred_element_type=f32)
    u1a = jnp.dot(x, a1ta_ref[...], preferred_element_type=f32)
    u0b = jnp.dot(x, a0tb_ref[...], preferred_element_type=f32)
    u1b = jnp.dot(x, a1tb_ref[...], preferred_element_type=f32)
    v2a = v2a_ref[...]
    v2b = v2b_ref[...]
    u0cat_ref[...] = jnp.concatenate([u0a, u0b], axis=1)
    u1s_ref[0:N, :] = jnp.concatenate(
        [u1a, u1b, _colvec(u1a, v2a), _colvec(u1b, v2b),
         jnp.zeros((N, DG - DX - 2), f32)], axis=1)
    u1s_ref[N:NP, :] = jnp.zeros((NP - N, DG), f32)
    ptg_ref[0:N, :] = jnp.concatenate(
        [_colvec(u0a, v2a), _colvec(u0b, v2b), jnp.zeros((N, DX - 2), f32)],
        axis=1)
    ptg_ref[N:NP, :] = jnp.zeros((NP - N, DX), f32)
    relp = relp_ref[...]
    ra = jnp.dot(relp, a2ta_ref[...], preferred_element_type=f32)
    rb = jnp.dot(relp, a2tb_ref[...], preferred_element_type=f32)
    rext_ref[...] = jnp.concatenate(
        [ra, rb, _colvec(ra, v2a), _colvec(rb, v2b),
         jnp.zeros((NRELP, DG - DX - 2), f32)], axis=1)
    o1 = jnp.dot(relp, w1_ref[...], preferred_element_type=f32)
    rl = jnp.dot(o1, a2lt_ref[...], preferred_element_type=f32)
    a2l = a2l_ref[...]
    rlext_ref[...] = jnp.concatenate(
        [rl, _colvec(rl, a2l), jnp.zeros((NRELP, DG - DX - 1), f32)], axis=1)
    orf_ref[...] = jnp.dot(o1, w3_ref[...], preferred_element_type=f32)


def _tc_mid(u0cat_ref, tcat_ref, sa_ref, sb_ref, ct_ref, cs_ref, ese_ref,
            a0lt_ref, a1lt_ref, a2l_ref,
            u0l_ref, u1sl_ref, ptgl_ref, tmask_ref):
    f32 = jnp.float32
    sa = sa_ref[0, :] + sa_ref[1, :]
    sb = sb_ref[0, :] + sb_ref[1, :]
    t = tcat_ref[0][0:N, :] + tcat_ref[1][0:N, :]
    scat = jnp.concatenate(
        [jnp.broadcast_to(sa[:, None], (N, DH)),
         jnp.broadcast_to(sb[:, None], (N, DH))], axis=1)
    h = (u0cat_ref[...] * scat + t) / (scat + 1e-12)
    x = _elu(h)
    ct = ct_ref[0, :] + ct_ref[1, :]
    cs = cs_ref[0, :] + cs_ref[1, :]
    tmask = ct > 0.5
    srcof = jnp.where(jnp.logical_and(cs > 0.5, jnp.logical_not(tmask)), 1.0, 0.0)
    x = jnp.where(srcof[:, None] > 0.5, ese_ref[...], x)
    u0l = jnp.dot(x, a0lt_ref[...], preferred_element_type=f32)
    u1l = jnp.dot(x, a1lt_ref[...], preferred_element_type=f32)
    u0l_ref[...] = u0l
    a2l = a2l_ref[...]
    u1sl_ref[0:N, :] = jnp.concatenate(
        [u1l, _colvec(u1l, a2l), jnp.zeros((N, DG - DX - 1), f32)], axis=1)
    u1sl_ref[N:NP, :] = jnp.zeros((NP - N, DG), f32)
    ptgl_ref[0:N, :] = jnp.concatenate(
        [_colvec(u0l, a2l), jnp.zeros((N, DX - 1), f32)], axis=1)
    ptgl_ref[N:NP, :] = jnp.zeros((NP - N, DX), f32)
    tmask_ref[...] = jnp.where(tmask, 1.0, 0.0)[None, :]


def _tc_fin(x_ref, u0l_ref, tl_ref, sl_ref, tmask_ref, ete_ref, out_ref):
    sl = sl_ref[0, :] + sl_ref[1, :]
    t = tl_ref[0][0:N, :] + tl_ref[1][0:N, :]
    h = (u0l_ref[...] * sl[:, None] + t) / (sl + 1e-12)[:, None]
    xf = _elu(h)
    tm = tmask_ref[0, :]
    out_ref[...] = jnp.where(tm[:, None] > 0.5, xf + ete_ref[...], x_ref[...])


_MESH = plsc.VectorSubcoreMesh(core_axis_name="c", subcore_axis_name="s",
                               num_cores=NC)


def _edge_pass(nh, tgt_hbm, src_hbm, et_hbm, u1s_hbm, rext_hbm, ptg_hbm,
               t_out, s_outs, tacc, saccs, scr):
    c = lax.axis_index("c")
    s = lax.axis_index("s")
    wid = c * NS + s
    njh = DX // (16 * nh)
    sets = [scr[b * 9:(b + 1) * 9] for b in range(2)]
    zb, zb1 = scr[18:20]
    isems = scr[20:22]
    gsems = scr[22:24]
    ssems = scr[24:26]

    iota16 = lax.iota(jnp.int32, 16)
    l0 = iota16 * 0
    l1 = l0 + 1
    ones16 = l0.astype(jnp.float32) + 1.0

    def zrow(i, carry):
        for j in range(DX // 16):
            zb[i, pl.ds(16 * j, 16)] = jnp.zeros((16,), jnp.float32)
        return carry
    lax.fori_loop(0, ZR, zrow, 0)

    def zrow1(i, carry):
        zb1[pl.ds(16 * i, 16)] = jnp.zeros((16,), jnp.float32)
        return carry
    lax.fori_loop(0, 63, zrow1, 0)

    for k in range(NZC):
        pltpu.sync_copy(zb, tacc.at[pl.ds(NR * s + ZR * k, ZR)])

    @pl.when(s == 0)
    def _():
        pltpu.sync_copy(zb, tacc.at[pl.ds(NR * NS, NTAIL)])

    @pl.when(s < 10)
    def _():
        for a in saccs:
            pltpu.sync_copy(zb1.at[pl.ds(0, 1000)], a.at[pl.ds(1000 * s, 1000)])
    plsc.subcore_barrier()

    for b in range(2):
        vfb = sets[b][8]
        for g in range(CH // 16):
            vfb[pl.ds(16 * g, 16)] = ones16

    def fire_idx(b, ci):
        tgt_v, src_v, et_v = sets[b][0:3]
        base = wid * EPW + ci * CH
        pltpu.async_copy(tgt_hbm.at[pl.ds(base, CH)], tgt_v, isems[b])
        pltpu.async_copy(src_hbm.at[pl.ds(base, CH)], src_v, isems[b])
        pltpu.async_copy(et_hbm.at[pl.ds(base, CH)], et_v, isems[b])

    def wait_idx(b):
        tgt_v, src_v, et_v = sets[b][0:3]
        pltpu.make_async_copy(tgt_hbm.at[pl.ds(0, CH)], tgt_v, isems[b]).wait()
        pltpu.make_async_copy(src_hbm.at[pl.ds(0, CH)], src_v, isems[b]).wait()
        pltpu.make_async_copy(et_hbm.at[pl.ds(0, CH)], et_v, isems[b]).wait()

    def fire_gathers(b):
        tgt_v, src_v, et_v, u1r, rr, scb = sets[b][0:6]
        pltpu.async_copy(u1s_hbm.at[src_v], u1r, gsems[b])
        pltpu.async_copy(rext_hbm.at[et_v], rr, gsems[b])
        pltpu.async_copy(ptg_hbm.at[tgt_v], scb, gsems[b])

    def wait_gathers(b):
        tgt_v, src_v, et_v, u1r, rr, scb = sets[b][0:6]
        pltpu.make_async_copy(u1s_hbm.at[src_v], u1r, gsems[b]).wait()
        pltpu.make_async_copy(rext_hbm.at[et_v], rr, gsems[b]).wait()
        pltpu.make_async_copy(ptg_hbm.at[tgt_v], scb, gsems[b]).wait()

    def fire_scatters(b):
        tgt_v, src_v, et_v, u1r, rr, scb, wa, wb, vf = sets[b]
        pltpu.async_copy(scb, tacc.at[tgt_v], ssems[b], add=True)
        pltpu.async_copy(wa.at[pl.ds(0, CH)], saccs[0].at[tgt_v], ssems[b],
                         add=True)
        if nh == 2:
            pltpu.async_copy(wb.at[pl.ds(0, CH)], saccs[1].at[tgt_v], ssems[b],
                             add=True)
            pltpu.async_copy(vf, saccs[2].at[tgt_v], ssems[b], add=True)
            pltpu.async_copy(vf, saccs[3].at[src_v], ssems[b], add=True)

    def wait_scatters(b):
        tgt_v, src_v, et_v, u1r, rr, scb, wa, wb, vf = sets[b]
        pltpu.make_async_copy(scb, tacc.at[tgt_v], ssems[b]).wait()
        pltpu.make_async_copy(wa.at[pl.ds(0, CH)], saccs[0].at[tgt_v],
                              ssems[b]).wait()
        if nh == 2:
            pltpu.make_async_copy(wb.at[pl.ds(0, CH)], saccs[1].at[tgt_v],
                                  ssems[b]).wait()
            pltpu.make_async_copy(vf, saccs[2].at[tgt_v], ssems[b]).wait()
            pltpu.make_async_copy(vf, saccs[3].at[src_v], ssems[b]).wait()

    def compute(b):
        tgt_v, src_v, et_v, u1r, rr, scb, wa, wb, vf = sets[b]

        def edge_body(i, carry2):
            sv = (scb[i, pl.ds(0, 16)] + u1r[i, pl.ds(DX, 16)]
                  + rr[i, pl.ds(DX, 16)])
            pw = jnp.where(sv > 0, -sv, (-ALPHA) * sv)
            wv = jnp.exp(pw)
            spl = [_splat(wv, l0)]
            wa[pl.ds(i, 16)] = spl[0]
            if nh == 2:
                spl.append(_splat(wv, l1))
                wb[pl.ds(i, 16)] = spl[1]
            for h in range(nh):
                for j in range(njh):
                    jsl = pl.ds(16 * (h * njh + j), 16)
                    scb[i, jsl] = spl[h] * (u1r[i, jsl] + rr[i, jsl])
            return carry2

        lax.fori_loop(0, CH, edge_body, 0)

    fire_idx(0, 0)
    fire_idx(1, 1)
    wait_idx(0)
    fire_gathers(0)
    wait_idx(1)
    fire_gathers(1)

    npair = NCHUNK // 2

    def pair_body(i, carry):
        e0 = 2 * i
        wait_gathers(0)
        compute(0)
        fire_scatters(0)
        wait_gathers(1)
        compute(1)
        fire_scatters(1)

        @pl.when(i + 1 < npair)
        def _():
            wait_scatters(0)
            fire_idx(0, e0 + 2)
            wait_idx(0)
            fire_gathers(0)
            wait_scatters(1)
            fire_idx(1, e0 + 3)
            wait_idx(1)
            fire_gathers(1)
        return carry

    lax.fori_loop(0, npair, pair_body, 0)
    wait_scatters(0)
    wait_scatters(1)
    plsc.subcore_barrier()

    for k in range(NZC):
        off = NR * s + ZR * k
        pltpu.sync_copy(tacc.at[pl.ds(off, ZR)], zb)
        pltpu.sync_copy(zb, t_out.at[c, pl.ds(off, ZR)])

    @pl.when(s == 0)
    def _():
        pltpu.sync_copy(tacc.at[pl.ds(NR * NS, NTAIL)], zb)
        pltpu.sync_copy(zb, t_out.at[c, pl.ds(NR * NS, NTAIL)])

    @pl.when(s < 10)
    def _():
        for a, o in zip(saccs, s_outs):
            pltpu.sync_copy(a.at[pl.ds(1000 * s, 1000)], zb1.at[pl.ds(0, 1000)])
            pltpu.sync_copy(zb1.at[pl.ds(0, 1000)],
                            o.at[pl.ds(c * N + 1000 * s, 1000)])


def _scratch_types(nsc):
    one_set = ([pltpu.VMEM((CH,), jnp.int32)] * 3 +
               [pltpu.VMEM((CH, DG), jnp.float32)] * 2 +
               [pltpu.VMEM((CH, DX), jnp.float32)] +
               [pltpu.VMEM((CH + 16,), jnp.float32)] * 2 +
               [pltpu.VMEM((CH,), jnp.float32)])
    return (one_set * 2 +
            [pltpu.VMEM((ZR, DX), jnp.float32),
             pltpu.VMEM((1008,), jnp.float32)] +
            [pltpu.SemaphoreType.DMA] * 6)


@functools.partial(
    pl.kernel,
    out_type=[jax.ShapeDtypeStruct((NC, NP, DX), jnp.float32)] +
             [jax.ShapeDtypeStruct((NC * N,), jnp.float32)] * 4,
    mesh=_MESH,
    scratch_types=[pltpu.VMEM_SHARED((NP, DX), jnp.float32)] +
                  [pltpu.VMEM_SHARED((NP,), jnp.float32)] * 4 +
                  _scratch_types(4),
)
def _sc_layer0(tgt_hbm, src_hbm, et_hbm, u1s_hbm, rext_hbm, ptg_hbm,
               t_out, sa_out, sb_out, ct_out, cs_out,
               tacc, sacc_a, sacc_b, ctacc, csacc, *scr):
    _edge_pass(2, tgt_hbm, src_hbm, et_hbm, u1s_hbm, rext_hbm, ptg_hbm,
               t_out, [sa_out, sb_out, ct_out, cs_out],
               tacc, [sacc_a, sacc_b, ctacc, csacc], list(scr))


@functools.partial(
    pl.kernel,
    out_type=[jax.ShapeDtypeStruct((NC, NP, DX), jnp.float32),
              jax.ShapeDtypeStruct((NC * N,), jnp.float32)],
    mesh=_MESH,
    scratch_types=[pltpu.VMEM_SHARED((NP, DX), jnp.float32),
                   pltpu.VMEM_SHARED((NP,), jnp.float32)] +
                  _scratch_types(1),
)
def _sc_final(tgt_hbm, src_hbm, et_hbm, u1s_hbm, rext_hbm, ptg_hbm,
              tl_out, sl_out, tacc, sacc, *scr):
    _edge_pass(1, tgt_hbm, src_hbm, et_hbm, u1s_hbm, rext_hbm, ptg_hbm,
               tl_out, [sl_out], tacc, [sacc], list(scr))


def kernel(entity_embeddings, relation_embed, edge_list, edge_type,
           W1, W3, W_source, W_target, a0, a2_0, a_last, a2_last):
    f32 = jnp.float32
    x = entity_embeddings.astype(f32)
    relp = jnp.zeros((NRELP, DRL), f32).at[:NREL].set(relation_embed.astype(f32))

    el = jnp.asarray(edge_list, jnp.int32)
    et = jnp.asarray(edge_type, jnp.int32)
    tgt = jnp.pad(el[0], (0, EPAD - NE), constant_values=N)
    src = jnp.pad(el[1], (0, EPAD - NE), constant_values=N)
    etp = jnp.pad(et, (0, EPAD - NE), constant_values=NREL)

    a0 = a0.astype(f32)
    a0ta = a0[0, :, :DX].T
    a1ta = a0[0, :, DX:2 * DX].T
    a2ta = a0[0, :, 2 * DX:].T
    a0tb = a0[1, :, :DX].T
    a1tb = a0[1, :, DX:2 * DX].T
    a2tb = a0[1, :, 2 * DX:].T
    v2a = a2_0[0].astype(f32)
    v2b = a2_0[1].astype(f32)
    a_last = a_last.astype(f32)
    a0lt = a_last[:, :DX].T
    a1lt = a_last[:, DX:2 * DX].T
    a2lt = a_last[:, 2 * DX:].T
    a2l = a2_last.astype(f32)

    shp = [
        jax.ShapeDtypeStruct((N, DX), f32),
        jax.ShapeDtypeStruct((N, DX), f32),
        jax.ShapeDtypeStruct((N, DX), f32),
        jax.ShapeDtypeStruct((NP, DG), f32),
        jax.ShapeDtypeStruct((NP, DX), f32),
        jax.ShapeDtypeStruct((NRELP, DG), f32),
        jax.ShapeDtypeStruct((NRELP, DG), f32),
        jax.ShapeDtypeStruct((NRELP, DX), f32),
    ]
    (ete, ese, u0cat, u1s, ptg, rext, rlext, orf) = pl.pallas_call(
        _tc_pre, out_shape=shp)(
        x, relp, W_target.astype(f32), W_source.astype(f32),
        a0ta, a1ta, a0tb, a1tb, a2ta, a2tb, v2a, v2b,
        W1.astype(f32), a2lt, a2l, W3.astype(f32))

    tcat, sa, sb, ct, cs = _sc_layer0(tgt, src, etp, u1s, rext, ptg)

    u0l, u1sl, ptgl, tmask = pl.pallas_call(
        _tc_mid,
        out_shape=[
            jax.ShapeDtypeStruct((N, DX), f32),
            jax.ShapeDtypeStruct((NP, DG), f32),
            jax.ShapeDtypeStruct((NP, DX), f32),
            jax.ShapeDtypeStruct((1, N), f32),
        ])(u0cat, tcat, sa.reshape(NC, N), sb.reshape(NC, N),
           ct.reshape(NC, N), cs.reshape(NC, N), ese, a0lt, a1lt, a2l)

    tl, sl = _sc_final(tgt, src, etp, u1sl, rlext, ptgl)

    new_emb = pl.pallas_call(
        _tc_fin, out_shape=jax.ShapeDtypeStruct((N, DX), f32))(
        x, u0l, tl, sl.reshape(NC, N), tmask, ete)

    return new_emb, orf[:NREL]
